# baseline (speedup 1.0000x reference)
import functools

import jax
import jax.numpy as jnp
from jax import lax
from jax.experimental import pallas as pl
from jax.experimental.pallas import tpu as pltpu
from jax.experimental.pallas import tpu_sc as plsc

N_NODES = 10000
N_EDGES = 320000
D_FEAT = 128

NC = 2
NS = 16
NW = NC * NS
L = 16

CHUNK = 128
NCHUNK = 2 * (-(-N_EDGES // (NW * CHUNK * 2)))
E_PAD = NW * NCHUNK * CHUNK
ROWS_PER_TILE = (N_NODES // NS) // 8 * 8
ROWS_REM = N_NODES - NS * ROWS_PER_TILE

_mesh = plsc.VectorSubcoreMesh(
    core_axis_name="c", subcore_axis_name="s", num_cores=NC, num_subcores=NS
)


@functools.partial(
    pl.kernel,
    out_type=jax.ShapeDtypeStruct((NC, N_NODES, D_FEAT), jnp.float32),
    mesh=_mesh,
    scratch_types=[
        pltpu.VMEM((CHUNK, D_FEAT), jnp.float32),
        pltpu.VMEM((CHUNK, D_FEAT), jnp.float32),
        pltpu.VMEM((3, CHUNK), jnp.int32),
        pltpu.VMEM((3, CHUNK), jnp.int32),
        pltpu.VMEM((3, CHUNK), jnp.int32),
        pltpu.VMEM_SHARED((N_NODES, D_FEAT), jnp.float32),
        pltpu.SemaphoreType.DMA,
        pltpu.SemaphoreType.DMA,
        pltpu.SemaphoreType.DMA,
        pltpu.SemaphoreType.DMA,
        pltpu.SemaphoreType.DMA,
        pltpu.SemaphoreType.DMA,
        pltpu.SemaphoreType.DMA,
    ],
)
def _sc_aggregate(ir_h, feats_h, partial_h,
                  gbuf0, gbuf1, ibuf0, ibuf1, ibuf2, acc,
                  gsem0, gsem1, ssem0, ssem1, isem0, isem1, isem2):
    cid = lax.axis_index("c")
    sid = lax.axis_index("s")
    wid = cid * NS + sid

    zero16 = jnp.zeros((L,), jnp.float32)

    def _zrow(i, carry):
        for r in range(D_FEAT // L):
            gbuf0[i, pl.ds(r * L, L)] = zero16
        return carry

    lax.fori_loop(0, CHUNK, _zrow, 0)
    base = sid * ROWS_PER_TILE
    full, rem = divmod(ROWS_PER_TILE, CHUNK)
    for k in range(full):
        pltpu.sync_copy(gbuf0, acc.at[pl.ds(base + k * CHUNK, CHUNK)])
    if rem:
        pltpu.sync_copy(gbuf0.at[pl.ds(0, rem)],
                        acc.at[pl.ds(base + full * CHUNK, rem)])

    @pl.when(sid == NS - 1)
    def _zero_tail():
        pltpu.sync_copy(gbuf0.at[pl.ds(0, ROWS_REM)],
                        acc.at[pl.ds(NS * ROWS_PER_TILE, ROWS_REM)])

    plsc.subcore_barrier()

    bufs = (gbuf0, gbuf1)
    gsems = (gsem0, gsem1)
    ssems = (ssem0, ssem1)
    ibufs = (ibuf0, ibuf1, ibuf2)
    isems = (isem0, isem1, isem2)

    def _scale(b, i3, j):
        buf = bufs[b]
        wrow = ibufs[i3]

        def _group(g, carry):
            w16 = lax.bitcast_convert_type(
                wrow[2, pl.ds(g * L, L)], jnp.float32)
            for i in range(L):
                e = g * L + i
                wi = jnp.broadcast_to(w16[i], (L,))
                for r in range(D_FEAT // L):
                    buf[e, pl.ds(r * L, L)] = buf[e, pl.ds(r * L, L)] * wi
            return carry

        lax.fori_loop(0, CHUNK // L, _group, 0)

    def _start_fetch(j, i3):
        pltpu.async_copy(ir_h.at[wid, j], ibufs[i3], isems[i3])

    def _wait_fetch(i3):
        pltpu.make_async_copy(ir_h.at[wid, 0], ibufs[i3], isems[i3]).wait()

    def _start_gather(b, i3):
        pltpu.async_copy(feats_h.at[ibufs[i3].at[0]], bufs[b], gsems[b])

    def _wait_gather(b, i3):
        pltpu.make_async_copy(feats_h.at[ibufs[i3].at[0]], bufs[b],
                              gsems[b]).wait()

    def _start_scatter(b, i3):
        pltpu.async_copy(bufs[b], acc.at[ibufs[i3].at[1]], ssems[b], add=True)

    def _wait_scatter(b):
        pltpu.make_async_copy(bufs[b], acc.at[ibufs[0].at[1]],
                              ssems[b]).wait()

    _start_fetch(0, 0)
    _start_fetch(1, 1)
    _wait_fetch(0)
    _start_gather(0, 0)

    _wait_gather(0, 0)
    _wait_fetch(1)
    _start_gather(1, 1)
    _start_fetch(2, 2)
    _scale(0, 0, 0)
    _start_scatter(0, 0)

    _wait_gather(1, 1)
    _wait_scatter(0)
    _wait_fetch(2)
    _start_gather(0, 2)
    _start_fetch(3, 0)
    _scale(1, 1, 1)
    _start_scatter(1, 1)

    def _six(t, carry):
        for k in range(6):
            j = 6 * t + 2 + k
            b = k % 2
            i3 = (2 + k) % 3
            _wait_gather(b, i3)
            _wait_scatter(1 - b)

            @pl.when(j + 1 < NCHUNK)
            def _next_gather():
                _wait_fetch((i3 + 1) % 3)
                _start_gather(1 - b, (i3 + 1) % 3)

            @pl.when(j + 2 < NCHUNK)
            def _next_fetch():
                _start_fetch(j + 2, (i3 + 2) % 3)

            _scale(b, i3, j)
            _start_scatter(b, i3)
        return carry

    lax.fori_loop(0, (NCHUNK - 2) // 6, _six, 0)

    _wait_scatter((NCHUNK - 1) % 2)

    plsc.subcore_barrier()
    pltpu.sync_copy(acc.at[pl.ds(base, ROWS_PER_TILE)],
                    partial_h.at[cid, pl.ds(base, ROWS_PER_TILE)])

    @pl.when(sid == NS - 1)
    def _flush_tail():
        pltpu.sync_copy(
            acc.at[pl.ds(NS * ROWS_PER_TILE, ROWS_REM)],
            partial_h.at[cid, pl.ds(NS * ROWS_PER_TILE, ROWS_REM)])


def _combine_body(p_ref, o_ref):
    o_ref[...] = p_ref[0] + p_ref[1]


_ROWS_BLK = 1000


@jax.jit
def _combine(partial):
    return pl.pallas_call(
        _combine_body,
        out_shape=jax.ShapeDtypeStruct((N_NODES, D_FEAT), jnp.float32),
        grid=(N_NODES // _ROWS_BLK,),
        in_specs=[pl.BlockSpec((NC, _ROWS_BLK, D_FEAT), lambda i: (0, i, 0))],
        out_specs=pl.BlockSpec((_ROWS_BLK, D_FEAT), lambda i: (i, 0)),
    )(partial)


@jax.jit
def kernel(edge_index, edge_weight, feats):
    rows = edge_index[0].astype(jnp.int32)
    cols = edge_index[1].astype(jnp.int32)
    w = lax.bitcast_convert_type(edge_weight.astype(jnp.float32), jnp.int32)

    pad = E_PAD - N_EDGES
    rows = jnp.concatenate([rows, jnp.zeros((pad,), jnp.int32)])
    cols = jnp.concatenate([cols, jnp.zeros((pad,), jnp.int32)])
    w = jnp.concatenate([w, jnp.zeros((pad,), jnp.int32)])

    ir = jnp.stack([cols.reshape(NW, NCHUNK, CHUNK),
                    rows.reshape(NW, NCHUNK, CHUNK),
                    w.reshape(NW, NCHUNK, CHUNK)], axis=2)

    partial = _sc_aggregate(ir, feats)
    return _combine(partial)
